# baseline (speedup 1.0000x reference)
import jax
import jax.numpy as jnp
from jax.experimental import pallas as pl
from jax.experimental.pallas import tpu as pltpu


def _gx0_kernel(idx_ref, emb_ref, wx_ref, b_ref, o_ref, ew_sc):
    @pl.when(pl.program_id(1) == 0)
    def _():
        ew_sc[...] = jnp.dot(emb_ref[...], wx_ref[...],
                             preferred_element_type=jnp.float32)

    V = emb_ref.shape[0]
    idx = idx_ref[0]
    iota_v = jax.lax.broadcasted_iota(jnp.int32, (V, idx.shape[1]), 0)
    oh_t = (iota_v == idx).astype(jnp.float32)
    gx = jax.lax.dot_general(
        oh_t, ew_sc[...],
        dimension_numbers=(((0,), (0,)), ((), ())),
        preferred_element_type=jnp.float32)
    o_ref[...] = gx + b_ref[...]


def _mm_bias_kernel(x_ref, w_ref, b_ref, o_ref):
    o_ref[...] = jnp.dot(x_ref[...], w_ref[...],
                         preferred_element_type=jnp.float32) + b_ref[...]


def _recur_kernel(gx_ref, wh_ref, h0_ref, c0_ref, hout_ref, cfin_ref,
                  h_sc, c_sc):
    H = h_sc.shape[1]
    t = pl.program_id(1)

    @pl.when(t == 0)
    def _():
        h_sc[...] = h0_ref[...]
        c_sc[...] = c0_ref[...]

    h = h_sc[...]
    c = c_sc[...]
    g = jnp.dot(h, wh_ref[...],
                preferred_element_type=jnp.float32) + gx_ref[0]
    sg_if = jax.nn.sigmoid(g[:, :2 * H])
    g_g = jnp.tanh(g[:, 2 * H:3 * H])
    o_g = jax.nn.sigmoid(g[:, 3 * H:])
    i_g = sg_if[:, :H]
    f_g = sg_if[:, H:]
    c_new = f_g * c + i_g * g_g
    h_new = o_g * jnp.tanh(c_new)
    h_sc[...] = h_new
    c_sc[...] = c_new
    hout_ref[...] = h_new[None]

    @pl.when(t == pl.num_programs(1) - 1)
    def _():
        cfin_ref[...] = c_new


def _batched_mm(x, w, b, mb):
    M, K = x.shape
    N = w.shape[1]
    nch = M // mb
    return pl.pallas_call(
        _mm_bias_kernel,
        grid=(2, nch // 2),
        in_specs=[
            pl.BlockSpec((mb, K), lambda bi, j, n=nch // 2: (bi * n + j, 0)),
            pl.BlockSpec((K, N), lambda bi, j: (0, 0)),
            pl.BlockSpec((1, N), lambda bi, j: (0, 0)),
        ],
        out_specs=pl.BlockSpec((mb, N), lambda bi, j, n=nch // 2: (bi * n + j, 0)),
        out_shape=jax.ShapeDtypeStruct((M, N), jnp.float32),
        compiler_params=pltpu.CompilerParams(
            dimension_semantics=("parallel", "arbitrary")),
    )(x, w, b)


def _recurrence(gx, wh, h0_l, c0_l):
    T, B, G = gx.shape
    H = G // 4
    Bb = B // 2
    h_all, c_fin = pl.pallas_call(
        _recur_kernel,
        grid=(2, T),
        in_specs=[
            pl.BlockSpec((1, Bb, G), lambda bi, t: (t, bi, 0)),
            pl.BlockSpec((H, G), lambda bi, t: (0, 0)),
            pl.BlockSpec((Bb, H), lambda bi, t: (bi, 0)),
            pl.BlockSpec((Bb, H), lambda bi, t: (bi, 0)),
        ],
        out_specs=[
            pl.BlockSpec((1, Bb, H), lambda bi, t: (t, bi, 0)),
            pl.BlockSpec((Bb, H), lambda bi, t: (bi, 0)),
        ],
        out_shape=[jax.ShapeDtypeStruct((T, B, H), jnp.float32),
                   jax.ShapeDtypeStruct((B, H), jnp.float32)],
        scratch_shapes=[pltpu.VMEM((Bb, H), jnp.float32),
                        pltpu.VMEM((Bb, H), jnp.float32)],
        compiler_params=pltpu.CompilerParams(
            dimension_semantics=("parallel", "arbitrary")),
    )(gx, wh, h0_l, c0_l)
    return h_all, c_fin


def kernel(idx_seq, emb, w_all, b_all, h0, c0):
    T, B = idx_seq.shape
    V, H = emb.shape
    L = w_all.shape[0] - 1
    G = 4 * H
    O = 256
    TB = T * B
    MB = 512 if TB % 1024 == 0 else TB // 2

    wx = [w_all[l, :H, :] for l in range(L)]
    wh = [w_all[l, H:, :] for l in range(L)]
    bl = [b_all[l] for l in range(L)]
    wdec = w_all[L, :H, :O]
    bdec = b_all[L, :, :O]

    nch = TB // MB
    idx_r = idx_seq.reshape(nch, 1, MB).astype(jnp.int32)

    gx0 = pl.pallas_call(
        _gx0_kernel,
        grid=(2, nch // 2),
        in_specs=[
            pl.BlockSpec((1, 1, MB), lambda bi, j, n=nch // 2: (bi * n + j, 0, 0)),
            pl.BlockSpec((V, H), lambda bi, j: (0, 0)),
            pl.BlockSpec((H, G), lambda bi, j: (0, 0)),
            pl.BlockSpec((1, G), lambda bi, j: (0, 0)),
        ],
        out_specs=pl.BlockSpec((MB, G), lambda bi, j, n=nch // 2: (bi * n + j, 0)),
        out_shape=jax.ShapeDtypeStruct((TB, G), jnp.float32),
        scratch_shapes=[pltpu.VMEM((V, G), jnp.float32)],
        compiler_params=pltpu.CompilerParams(
            dimension_semantics=("parallel", "arbitrary")),
    )(idx_r, emb, wx[0], bl[0])

    h_all0, c_fin0 = _recurrence(gx0.reshape(T, B, G), wh[0], h0[0], c0[0])

    gx1 = _batched_mm(h_all0.reshape(TB, H), wx[1], bl[1], MB)

    h_all1, c_fin1 = _recurrence(gx1.reshape(T, B, G), wh[1], h0[1], c0[1])

    logits = _batched_mm(h_all1.reshape(TB, H), wdec, bdec, MB)

    h_n = jnp.stack([h_all0[T - 1], h_all1[T - 1]])
    c_n = jnp.stack([c_fin0, c_fin1])
    return logits.reshape(T, B, O), (h_n, c_n)


# speedup vs baseline: 1.5903x; 1.5903x over previous
import jax
import jax.numpy as jnp
from jax.experimental import pallas as pl
from jax.experimental.pallas import tpu as pltpu


def _lstm_steps(wh_ref, gx_sc, hout_ref, h_sc, c_sc, TC):
    H = h_sc.shape[1]

    def step(t, carry):
        h, c = carry
        g = jnp.dot(h, wh_ref[0], preferred_element_type=jnp.float32) + gx_sc[t]
        sg_if = jax.nn.sigmoid(g[:, :2 * H])
        g_g = jnp.tanh(g[:, 2 * H:3 * H])
        o_g = jax.nn.sigmoid(g[:, 3 * H:])
        c_new = sg_if[:, H:] * c + sg_if[:, :H] * g_g
        h_new = o_g * jnp.tanh(c_new)
        hout_ref[t] = h_new
        return h_new, c_new

    h_f, c_f = jax.lax.fori_loop(0, TC, step, (h_sc[...], c_sc[...]))
    h_sc[...] = h_f
    c_sc[...] = c_f


def _l0_kernel(idx_ref, emb_ref, wx_ref, wh_ref, b_ref, h0_ref, c0_ref,
               hout_ref, cfin_ref, ew_sc, gx_sc, h_sc, c_sc):
    TC, Bb, H = hout_ref.shape
    V = emb_ref.shape[0]

    @pl.when(pl.program_id(1) == 0)
    def _():
        ew_sc[...] = jnp.dot(emb_ref[...], wx_ref[0],
                             preferred_element_type=jnp.float32)
        h_sc[...] = h0_ref[0]
        c_sc[...] = c0_ref[0]

    idx = idx_ref[0, 0]
    iota_v = jax.lax.broadcasted_iota(jnp.int32, (V, TC * Bb), 0)
    oh_t = (iota_v == idx).astype(jnp.float32)
    gx = jax.lax.dot_general(
        oh_t, ew_sc[...],
        dimension_numbers=(((0,), (0,)), ((), ())),
        preferred_element_type=jnp.float32) + b_ref[0]
    gx_sc[...] = gx.reshape(TC, Bb, 4 * H)

    _lstm_steps(wh_ref, gx_sc, hout_ref, h_sc, c_sc, TC)
    cfin_ref[...] = c_sc[...]


def _l1_kernel(hin_ref, wx_ref, wh_ref, b_ref, h0_ref, c0_ref,
               hout_ref, cfin_ref, gx_sc, h_sc, c_sc):
    TC, Bb, H = hin_ref.shape

    @pl.when(pl.program_id(1) == 0)
    def _():
        h_sc[...] = h0_ref[0]
        c_sc[...] = c0_ref[0]

    x = hin_ref[...].reshape(TC * Bb, H)
    gx = jnp.dot(x, wx_ref[0], preferred_element_type=jnp.float32) + b_ref[0]
    gx_sc[...] = gx.reshape(TC, Bb, 4 * H)

    _lstm_steps(wh_ref, gx_sc, hout_ref, h_sc, c_sc, TC)
    cfin_ref[...] = c_sc[...]


def _dec_kernel(x_ref, w_ref, b_ref, o_ref):
    o_ref[...] = jnp.dot(x_ref[...], w_ref[0],
                         preferred_element_type=jnp.float32) + b_ref[0]


def kernel(idx_seq, emb, w_all, b_all, h0, c0):
    T, B = idx_seq.shape
    V, H = emb.shape
    G = 4 * H
    O = 256
    TB = T * B
    TC = 16 if T % 16 == 0 else T
    NT = T // TC
    Bb = B // 2

    idx_r = (idx_seq.astype(jnp.int32)
             .reshape(NT, TC, 2, Bb).transpose(0, 2, 1, 3)
             .reshape(NT, 2, 1, TC * Bb))

    sem = pltpu.CompilerParams(dimension_semantics=("parallel", "arbitrary"))

    def layer_specs(l):
        return [
            pl.BlockSpec((1, H, G), lambda bi, j, l=l: (l, 0, 0)),
            pl.BlockSpec((1, H, G), lambda bi, j, l=l: (l, 1, 0)),
            pl.BlockSpec((1, 1, G), lambda bi, j, l=l: (l, 0, 0)),
            pl.BlockSpec((1, Bb, H), lambda bi, j, l=l: (l, bi, 0)),
            pl.BlockSpec((1, Bb, H), lambda bi, j, l=l: (l, bi, 0)),
        ]

    out_specs = [
        pl.BlockSpec((TC, Bb, H), lambda bi, j: (j, bi, 0)),
        pl.BlockSpec((Bb, H), lambda bi, j: (bi, 0)),
    ]
    out_shape = [jax.ShapeDtypeStruct((T, B, H), jnp.float32),
                 jax.ShapeDtypeStruct((B, H), jnp.float32)]
    state_scratch = [pltpu.VMEM((TC, Bb, G), jnp.float32),
                     pltpu.VMEM((Bb, H), jnp.float32),
                     pltpu.VMEM((Bb, H), jnp.float32)]

    h_all0, c_fin0 = pl.pallas_call(
        _l0_kernel,
        grid=(2, NT),
        in_specs=[pl.BlockSpec((1, 1, 1, TC * Bb), lambda bi, j: (j, bi, 0, 0)),
                  pl.BlockSpec((V, H), lambda bi, j: (0, 0))] + layer_specs(0),
        out_specs=out_specs,
        out_shape=out_shape,
        scratch_shapes=[pltpu.VMEM((V, G), jnp.float32)] + state_scratch,
        compiler_params=sem,
    )(idx_r, emb, w_all, w_all, b_all, h0, c0)

    h_all1, c_fin1 = pl.pallas_call(
        _l1_kernel,
        grid=(2, NT),
        in_specs=[pl.BlockSpec((TC, Bb, H), lambda bi, j: (j, bi, 0))]
                 + layer_specs(1),
        out_specs=out_specs,
        out_shape=out_shape,
        scratch_shapes=state_scratch,
        compiler_params=sem,
    )(h_all0, w_all, w_all, b_all, h0, c0)

    MBd = TB // 4
    logits = pl.pallas_call(
        _dec_kernel,
        grid=(2, 2),
        in_specs=[
            pl.BlockSpec((MBd, H), lambda bi, j: (bi * 2 + j, 0)),
            pl.BlockSpec((1, H, O), lambda bi, j: (2, 0, 0)),
            pl.BlockSpec((1, 1, O), lambda bi, j: (2, 0, 0)),
        ],
        out_specs=pl.BlockSpec((MBd, O), lambda bi, j: (bi * 2 + j, 0)),
        out_shape=jax.ShapeDtypeStruct((TB, O), jnp.float32),
        compiler_params=sem,
    )(h_all1.reshape(TB, H), w_all, b_all)

    h_n = jnp.stack([h_all0[T - 1], h_all1[T - 1]])
    c_n = jnp.stack([c_fin0, c_fin1])
    return logits.reshape(T, B, O), (h_n, c_n)


# speedup vs baseline: 2.6137x; 1.6435x over previous
import jax
import jax.numpy as jnp
from jax.experimental import pallas as pl
from jax.experimental.pallas import tpu as pltpu


def _sig(x):
    return 0.5 * jnp.tanh(0.5 * x) + 0.5


def _lstm_steps(wh_ref, gx_sc, hout_ref, h_sc, c_sc, TC):
    H = h_sc.shape[1]
    U = 4

    def group(gidx, carry):
        h, c = carry
        base = gidx * U
        for u in range(U):
            t = base + u
            g = jnp.dot(h, wh_ref[0],
                        preferred_element_type=jnp.float32) + gx_sc[t]
            sg_if = _sig(g[:, :2 * H])
            g_g = jnp.tanh(g[:, 2 * H:3 * H])
            o_g = _sig(g[:, 3 * H:])
            c = sg_if[:, H:] * c + sg_if[:, :H] * g_g
            h = o_g * jnp.tanh(c)
            hout_ref[t] = h
        return h, c

    h_f, c_f = jax.lax.fori_loop(0, TC // U, group, (h_sc[...], c_sc[...]))
    h_sc[...] = h_f
    c_sc[...] = c_f


def _l0_kernel(idx_ref, emb_ref, wx_ref, wh_ref, b_ref, h0_ref, c0_ref,
               hout_ref, cfin_ref, ew_sc, gx_sc, h_sc, c_sc):
    TC, Bf, H = hout_ref.shape
    V = emb_ref.shape[0]

    @pl.when(pl.program_id(0) == 0)
    def _():
        ew_sc[...] = jnp.dot(emb_ref[...], wx_ref[0],
                             preferred_element_type=jnp.float32)
        h_sc[...] = h0_ref[0]
        c_sc[...] = c0_ref[0]

    idx = idx_ref[0]
    iota_v = jax.lax.broadcasted_iota(jnp.int32, (V, TC * Bf), 0)
    oh_t = (iota_v == idx).astype(jnp.float32)
    gx = jax.lax.dot_general(
        oh_t, ew_sc[...],
        dimension_numbers=(((0,), (0,)), ((), ())),
        preferred_element_type=jnp.float32) + b_ref[0]
    gx_sc[...] = gx.reshape(TC, Bf, 4 * H)

    _lstm_steps(wh_ref, gx_sc, hout_ref, h_sc, c_sc, TC)
    cfin_ref[...] = c_sc[...]


def _l1_kernel(hin_ref, wx_ref, wh_ref, b_ref, h0_ref, c0_ref,
               hout_ref, cfin_ref, gx_sc, h_sc, c_sc):
    TC, Bf, H = hin_ref.shape

    @pl.when(pl.program_id(0) == 0)
    def _():
        h_sc[...] = h0_ref[0]
        c_sc[...] = c0_ref[0]

    x = hin_ref[...].reshape(TC * Bf, H)
    gx = jnp.dot(x, wx_ref[0], preferred_element_type=jnp.float32) + b_ref[0]
    gx_sc[...] = gx.reshape(TC, Bf, 4 * H)

    _lstm_steps(wh_ref, gx_sc, hout_ref, h_sc, c_sc, TC)
    cfin_ref[...] = c_sc[...]


def _dec_kernel(x_ref, w_ref, b_ref, o_ref):
    o_ref[...] = jnp.dot(x_ref[...], w_ref[0],
                         preferred_element_type=jnp.float32) + b_ref[0]


def kernel(idx_seq, emb, w_all, b_all, h0, c0):
    T, B = idx_seq.shape
    V, H = emb.shape
    G = 4 * H
    O = 256
    TB = T * B
    TC = 16 if T % 16 == 0 else T
    NT = T // TC

    idx_r = idx_seq.astype(jnp.int32).reshape(NT, 1, TC * B)

    def layer_specs(l):
        return [
            pl.BlockSpec((1, H, G), lambda j, l=l: (l, 0, 0)),
            pl.BlockSpec((1, H, G), lambda j, l=l: (l, 1, 0)),
            pl.BlockSpec((1, 1, G), lambda j, l=l: (l, 0, 0)),
            pl.BlockSpec((1, B, H), lambda j, l=l: (l, 0, 0)),
            pl.BlockSpec((1, B, H), lambda j, l=l: (l, 0, 0)),
        ]

    out_specs = [
        pl.BlockSpec((TC, B, H), lambda j: (j, 0, 0)),
        pl.BlockSpec((B, H), lambda j: (0, 0)),
    ]
    out_shape = [jax.ShapeDtypeStruct((T, B, H), jnp.float32),
                 jax.ShapeDtypeStruct((B, H), jnp.float32)]
    state_scratch = [pltpu.VMEM((TC, B, G), jnp.float32),
                     pltpu.VMEM((B, H), jnp.float32),
                     pltpu.VMEM((B, H), jnp.float32)]
    seq_sem = pltpu.CompilerParams(dimension_semantics=("arbitrary",))

    h_all0, c_fin0 = pl.pallas_call(
        _l0_kernel,
        grid=(NT,),
        in_specs=[pl.BlockSpec((1, 1, TC * B), lambda j: (j, 0, 0)),
                  pl.BlockSpec((V, H), lambda j: (0, 0))] + layer_specs(0),
        out_specs=out_specs,
        out_shape=out_shape,
        scratch_shapes=[pltpu.VMEM((V, G), jnp.float32)] + state_scratch,
        compiler_params=seq_sem,
    )(idx_r, emb, w_all, w_all, b_all, h0, c0)

    h_all1, c_fin1 = pl.pallas_call(
        _l1_kernel,
        grid=(NT,),
        in_specs=[pl.BlockSpec((TC, B, H), lambda j: (j, 0, 0))]
                 + layer_specs(1),
        out_specs=out_specs,
        out_shape=out_shape,
        scratch_shapes=state_scratch,
        compiler_params=seq_sem,
    )(h_all0, w_all, w_all, b_all, h0, c0)

    MBd = TB // 4
    logits = pl.pallas_call(
        _dec_kernel,
        grid=(2, 2),
        in_specs=[
            pl.BlockSpec((MBd, H), lambda bi, j: (bi * 2 + j, 0)),
            pl.BlockSpec((1, H, O), lambda bi, j: (2, 0, 0)),
            pl.BlockSpec((1, 1, O), lambda bi, j: (2, 0, 0)),
        ],
        out_specs=pl.BlockSpec((MBd, O), lambda bi, j: (bi * 2 + j, 0)),
        out_shape=jax.ShapeDtypeStruct((TB, O), jnp.float32),
        compiler_params=pltpu.CompilerParams(
            dimension_semantics=("parallel", "arbitrary")),
    )(h_all1.reshape(TB, H), w_all, b_all)

    h_n = jnp.stack([h_all0[T - 1], h_all1[T - 1]])
    c_n = jnp.stack([c_fin0, c_fin1])
    return logits.reshape(T, B, O), (h_n, c_n)


# speedup vs baseline: 2.7286x; 1.0439x over previous
import jax
import jax.numpy as jnp
from jax.experimental import pallas as pl
from jax.experimental.pallas import tpu as pltpu


def _sig(x):
    return 0.5 * jnp.tanh(0.5 * x) + 0.5


def _lstm_steps(wh_ref, gx_sc, hout_ref, h_sc, c_sc, TC):
    H = h_sc.shape[1]
    U = TC

    def group(gidx, carry):
        h, c = carry
        base = gidx * U
        for u in range(U):
            t = base + u
            g = jnp.dot(h, wh_ref[0],
                        preferred_element_type=jnp.float32) + gx_sc[t]
            sg_if = _sig(g[:, :2 * H])
            g_g = jnp.tanh(g[:, 2 * H:3 * H])
            o_g = _sig(g[:, 3 * H:])
            c = sg_if[:, H:] * c + sg_if[:, :H] * g_g
            h = o_g * jnp.tanh(c)
            hout_ref[t] = h
        return h, c

    h_f, c_f = jax.lax.fori_loop(0, TC // U, group, (h_sc[...], c_sc[...]))
    h_sc[...] = h_f
    c_sc[...] = c_f


def _l0_kernel(idx_ref, emb_ref, wx_ref, wh_ref, b_ref, h0_ref, c0_ref,
               hout_ref, cfin_ref, ew_sc, gx_sc, h_sc, c_sc):
    TC, Bf, H = hout_ref.shape
    V = emb_ref.shape[0]

    @pl.when(pl.program_id(0) == 0)
    def _():
        ew_sc[...] = jnp.dot(emb_ref[...], wx_ref[0],
                             preferred_element_type=jnp.float32)
        h_sc[...] = h0_ref[0]
        c_sc[...] = c0_ref[0]

    idx = idx_ref[0]
    iota_v = jax.lax.broadcasted_iota(jnp.int32, (V, TC * Bf), 0)
    oh_t = (iota_v == idx).astype(jnp.float32)
    gx = jax.lax.dot_general(
        oh_t, ew_sc[...],
        dimension_numbers=(((0,), (0,)), ((), ())),
        preferred_element_type=jnp.float32) + b_ref[0]
    gx_sc[...] = gx.reshape(TC, Bf, 4 * H)

    _lstm_steps(wh_ref, gx_sc, hout_ref, h_sc, c_sc, TC)
    cfin_ref[...] = c_sc[...]


def _l1_kernel(hin_ref, wx_ref, wh_ref, b_ref, h0_ref, c0_ref,
               hout_ref, cfin_ref, gx_sc, h_sc, c_sc):
    TC, Bf, H = hin_ref.shape

    @pl.when(pl.program_id(0) == 0)
    def _():
        h_sc[...] = h0_ref[0]
        c_sc[...] = c0_ref[0]

    x = hin_ref[...].reshape(TC * Bf, H)
    gx = jnp.dot(x, wx_ref[0], preferred_element_type=jnp.float32) + b_ref[0]
    gx_sc[...] = gx.reshape(TC, Bf, 4 * H)

    _lstm_steps(wh_ref, gx_sc, hout_ref, h_sc, c_sc, TC)
    cfin_ref[...] = c_sc[...]


def _dec_kernel(x_ref, w_ref, b_ref, o_ref):
    o_ref[...] = jnp.dot(x_ref[...], w_ref[0],
                         preferred_element_type=jnp.float32) + b_ref[0]


def kernel(idx_seq, emb, w_all, b_all, h0, c0):
    T, B = idx_seq.shape
    V, H = emb.shape
    G = 4 * H
    O = 256
    TB = T * B
    TC = 16 if T % 16 == 0 else T
    NT = T // TC

    idx_r = idx_seq.astype(jnp.int32).reshape(NT, 1, TC * B)

    def layer_specs(l):
        return [
            pl.BlockSpec((1, H, G), lambda j, l=l: (l, 0, 0)),
            pl.BlockSpec((1, H, G), lambda j, l=l: (l, 1, 0)),
            pl.BlockSpec((1, 1, G), lambda j, l=l: (l, 0, 0)),
            pl.BlockSpec((1, B, H), lambda j, l=l: (l, 0, 0)),
            pl.BlockSpec((1, B, H), lambda j, l=l: (l, 0, 0)),
        ]

    out_specs = [
        pl.BlockSpec((TC, B, H), lambda j: (j, 0, 0)),
        pl.BlockSpec((B, H), lambda j: (0, 0)),
    ]
    out_shape = [jax.ShapeDtypeStruct((T, B, H), jnp.float32),
                 jax.ShapeDtypeStruct((B, H), jnp.float32)]
    state_scratch = [pltpu.VMEM((TC, B, G), jnp.float32),
                     pltpu.VMEM((B, H), jnp.float32),
                     pltpu.VMEM((B, H), jnp.float32)]
    seq_sem = pltpu.CompilerParams(dimension_semantics=("arbitrary",))

    h_all0, c_fin0 = pl.pallas_call(
        _l0_kernel,
        grid=(NT,),
        in_specs=[pl.BlockSpec((1, 1, TC * B), lambda j: (j, 0, 0)),
                  pl.BlockSpec((V, H), lambda j: (0, 0))] + layer_specs(0),
        out_specs=out_specs,
        out_shape=out_shape,
        scratch_shapes=[pltpu.VMEM((V, G), jnp.float32)] + state_scratch,
        compiler_params=seq_sem,
    )(idx_r, emb, w_all, w_all, b_all, h0, c0)

    h_all1, c_fin1 = pl.pallas_call(
        _l1_kernel,
        grid=(NT,),
        in_specs=[pl.BlockSpec((TC, B, H), lambda j: (j, 0, 0))]
                 + layer_specs(1),
        out_specs=out_specs,
        out_shape=out_shape,
        scratch_shapes=state_scratch,
        compiler_params=seq_sem,
    )(h_all0, w_all, w_all, b_all, h0, c0)

    MBd = TB // 4
    logits = pl.pallas_call(
        _dec_kernel,
        grid=(2, 2),
        in_specs=[
            pl.BlockSpec((MBd, H), lambda bi, j: (bi * 2 + j, 0)),
            pl.BlockSpec((1, H, O), lambda bi, j: (2, 0, 0)),
            pl.BlockSpec((1, 1, O), lambda bi, j: (2, 0, 0)),
        ],
        out_specs=pl.BlockSpec((MBd, O), lambda bi, j: (bi * 2 + j, 0)),
        out_shape=jax.ShapeDtypeStruct((TB, O), jnp.float32),
        compiler_params=pltpu.CompilerParams(
            dimension_semantics=("parallel", "arbitrary")),
    )(h_all1.reshape(TB, H), w_all, b_all)

    h_n = jnp.stack([h_all0[T - 1], h_all1[T - 1]])
    c_n = jnp.stack([c_fin0, c_fin1])
    return logits.reshape(T, B, O), (h_n, c_n)
